# Initial kernel scaffold; baseline (speedup 1.0000x reference)
import jax
import jax.numpy as jnp
from jax.experimental import pallas as pl


def kernel(nodes, neigh_idx, features, weight):
    raise NotImplementedError("write your pallas kernel here")


# speedup vs baseline: 5.7471x; 5.7471x over previous
import functools

import jax
import jax.numpy as jnp
from jax import lax
from jax.experimental import pallas as pl
from jax.experimental.pallas import tpu as pltpu, tpu_sc as plsc

_B = 4096
_S = 32
_F = 128
_NW = 32
_BW = _B // _NW


def _sc_body(feat_hbm, nodes_hbm, neighT_hbm, loc_hbm,
             self_out, neigh_out,
             idx_s, nodes_v, loc_v, buf0, buf1, self_buf, acc_sh):
    c = lax.axis_index("c")
    q = lax.axis_index("s")
    w = c * 16 + q
    base = w * _BW

    pltpu.sync_copy(neighT_hbm.at[w], idx_s)
    pltpu.sync_copy(nodes_hbm.at[pl.ds(base, _BW)], nodes_v)
    pltpu.sync_copy(loc_hbm.at[pl.ds(base, _BW)], loc_v)

    pltpu.sync_copy(feat_hbm.at[nodes_v], self_buf)
    pltpu.sync_copy(self_buf, self_out.at[pl.ds(base, _BW)])

    pltpu.sync_copy(feat_hbm.at[idx_s.at[0]], buf0)
    pltpu.sync_copy(buf0, acc_sh.at[loc_v])

    def round_body(j, carry):
        pltpu.sync_copy(feat_hbm.at[idx_s.at[j]], buf1)
        pltpu.sync_copy(buf1, acc_sh.at[loc_v], add=True)
        return carry

    lax.fori_loop(1, _S, round_body, 0)

    lbase = q * _BW
    pltpu.sync_copy(acc_sh.at[pl.ds(lbase, _BW)], buf0)
    pltpu.sync_copy(buf0, neigh_out.at[pl.ds(base, _BW)])


def _sc_gather(features, nodes, neighTw, loc):
    mesh = plsc.VectorSubcoreMesh(core_axis_name="c", subcore_axis_name="s")
    f32 = jnp.float32
    return pl.kernel(
        _sc_body,
        out_type=(jax.ShapeDtypeStruct((_B, _F), f32),
                  jax.ShapeDtypeStruct((_B, _F), f32)),
        mesh=mesh,
        scratch_types=[
            pltpu.VMEM((_S, _BW), jnp.int32),
            pltpu.VMEM((_BW,), jnp.int32),
            pltpu.VMEM((_BW,), jnp.int32),
            pltpu.VMEM((_BW, _F), f32),
            pltpu.VMEM((_BW, _F), f32),
            pltpu.VMEM((_BW, _F), f32),
            pltpu.VMEM_SHARED((_B // 2, _F), f32),
        ],
    )(features, nodes, neighTw, loc)


def _tc_body(self_ref, neigh_ref, w_ref, out_ref):
    w1 = w_ref[0:_F, :]
    w2 = w_ref[_F:2 * _F, :] * (1.0 / _S)
    a = lax.dot_general(w1, self_ref[...], (((0,), (1,)), ((), ())),
                        preferred_element_type=jnp.float32)
    b = lax.dot_general(w2, neigh_ref[...], (((0,), (1,)), ((), ())),
                        preferred_element_type=jnp.float32)
    out_ref[...] = jnp.maximum(a + b, 0.0)


def _tc_project(self_feats, neigh_sum, weight):
    blk = 1024
    grid = (_B // blk,)
    return pl.pallas_call(
        _tc_body,
        grid=grid,
        in_specs=[
            pl.BlockSpec((blk, _F), lambda i: (i, 0)),
            pl.BlockSpec((blk, _F), lambda i: (i, 0)),
            pl.BlockSpec((2 * _F, _F), lambda i: (0, 0)),
        ],
        out_specs=pl.BlockSpec((_F, blk), lambda i: (0, i)),
        out_shape=jax.ShapeDtypeStruct((_F, _B), jnp.float32),
    )(self_feats, neigh_sum, weight)


@jax.jit
def kernel(nodes, neigh_idx, features, weight):
    nodes = nodes.astype(jnp.int32)
    neighTw = jnp.transpose(
        neigh_idx.astype(jnp.int32).reshape(_NW, _BW, _S), (0, 2, 1))
    loc = jnp.arange(_B, dtype=jnp.int32) % (_B // 2)
    self_feats, neigh_sum = _sc_gather(features, nodes, neighTw, loc)
    return _tc_project(self_feats, neigh_sum, weight)


# speedup vs baseline: 8.8145x; 1.5337x over previous
import functools

import jax
import jax.numpy as jnp
from jax import lax
from jax.experimental import pallas as pl
from jax.experimental.pallas import tpu as pltpu, tpu_sc as plsc

_B = 4096
_S = 32
_F = 128
_NW = 32
_BW = _B // _NW
_NBUF = 4


def _sc_body(feat_hbm, nodes_hbm, neighT_hbm, loc_hbm,
             self_out, neigh_out,
             idx_s, nodes_v, loc_v, b0, b1, b2, b3, self_buf, acc_sh,
             g0, g1, g2, g3, s0, s1, s2, s3, selfsem):
    bufs = (b0, b1, b2, b3)
    gsem = (g0, g1, g2, g3)
    ssem = (s0, s1, s2, s3)
    c = lax.axis_index("c")
    q = lax.axis_index("s")
    w = c * 16 + q
    base = w * _BW
    lbase = q * _BW

    pltpu.sync_copy(neighT_hbm.at[w], idx_s)
    pltpu.sync_copy(loc_hbm.at[pl.ds(base, _BW)], loc_v)

    def zero_row(r, carry):
        for cc in range(_F // 16):
            b0[r, pl.ds(cc * 16, 16)] = jnp.zeros((16,), jnp.float32)
        return carry

    lax.fori_loop(0, _BW, zero_row, 0)
    pltpu.sync_copy(b0, acc_sh.at[pl.ds(lbase, _BW)])

    pltpu.sync_copy(nodes_hbm.at[pl.ds(base, _BW)], nodes_v)
    pltpu.async_copy(feat_hbm.at[nodes_v], self_buf, selfsem)

    for b in range(_NBUF):
        pltpu.async_copy(feat_hbm.at[idx_s.at[b]], bufs[b], gsem[b])

    def wait_gather(b):
        pltpu.make_async_copy(feat_hbm.at[pl.ds(0, _BW)], bufs[b],
                              gsem[b]).wait()

    def wait_scatter(b):
        pltpu.make_async_copy(bufs[b], acc_sh.at[pl.ds(lbase, _BW)],
                              ssem[b]).wait()

    n_groups = _S // _NBUF

    def group(g, carry):
        for b in range(_NBUF):
            j = g * _NBUF + b
            wait_gather(b)
            pltpu.async_copy(bufs[b], acc_sh.at[loc_v], ssem[b], add=True)
            wait_scatter(b)
            pltpu.async_copy(feat_hbm.at[idx_s.at[j + _NBUF]], bufs[b],
                             gsem[b])
        return carry

    lax.fori_loop(0, n_groups - 1, group, 0)

    for b in range(_NBUF):
        wait_gather(b)
        pltpu.async_copy(bufs[b], acc_sh.at[loc_v], ssem[b], add=True)
    for b in range(_NBUF):
        wait_scatter(b)

    pltpu.make_async_copy(feat_hbm.at[pl.ds(0, _BW)], self_buf,
                          selfsem).wait()
    pltpu.sync_copy(self_buf, self_out.at[pl.ds(base, _BW)])
    pltpu.sync_copy(acc_sh.at[pl.ds(lbase, _BW)],
                    neigh_out.at[pl.ds(base, _BW)])


def _sc_gather(features, nodes, neighTw, loc):
    mesh = plsc.VectorSubcoreMesh(core_axis_name="c", subcore_axis_name="s")
    f32 = jnp.float32
    return pl.kernel(
        _sc_body,
        out_type=(jax.ShapeDtypeStruct((_B, _F), f32),
                  jax.ShapeDtypeStruct((_B, _F), f32)),
        mesh=mesh,
        scratch_types=[
            pltpu.VMEM((_S, _BW), jnp.int32),
            pltpu.VMEM((_BW,), jnp.int32),
            pltpu.VMEM((_BW,), jnp.int32),
        ] + [pltpu.VMEM((_BW, _F), f32) for _ in range(_NBUF)]
        + [
            pltpu.VMEM((_BW, _F), f32),
            pltpu.VMEM_SHARED((_B // 2, _F), f32),
        ] + [pltpu.SemaphoreType.DMA] * (2 * _NBUF + 1),
    )(features, nodes, neighTw, loc)


def _tc_body(self_ref, neigh_ref, w_ref, out_ref):
    w1 = w_ref[0:_F, :]
    w2 = w_ref[_F:2 * _F, :] * (1.0 / _S)
    a = lax.dot_general(w1, self_ref[...], (((0,), (1,)), ((), ())),
                        preferred_element_type=jnp.float32)
    b = lax.dot_general(w2, neigh_ref[...], (((0,), (1,)), ((), ())),
                        preferred_element_type=jnp.float32)
    out_ref[...] = jnp.maximum(a + b, 0.0)


def _tc_project(self_feats, neigh_sum, weight):
    blk = 1024
    grid = (_B // blk,)
    return pl.pallas_call(
        _tc_body,
        grid=grid,
        in_specs=[
            pl.BlockSpec((blk, _F), lambda i: (i, 0)),
            pl.BlockSpec((blk, _F), lambda i: (i, 0)),
            pl.BlockSpec((2 * _F, _F), lambda i: (0, 0)),
        ],
        out_specs=pl.BlockSpec((_F, blk), lambda i: (0, i)),
        out_shape=jax.ShapeDtypeStruct((_F, _B), jnp.float32),
    )(self_feats, neigh_sum, weight)


@jax.jit
def kernel(nodes, neigh_idx, features, weight):
    nodes = nodes.astype(jnp.int32)
    neighTw = jnp.transpose(
        neigh_idx.astype(jnp.int32).reshape(_NW, _BW, _S), (0, 2, 1))
    loc = jnp.arange(_B, dtype=jnp.int32) % (_B // 2)
    self_feats, neigh_sum = _sc_gather(features, nodes, neighTw, loc)
    return _tc_project(self_feats, neigh_sum, weight)
